# baseline (speedup 1.0000x reference)
import jax
import jax.numpy as jnp
from jax import lax
from jax.experimental import pallas as pl
from jax.experimental.pallas import tpu as pltpu
from jax.experimental.pallas import tpu_sc as plsc

N = 50000
E = 800000
EPAD = 819200
HID = 3
W = 7
PADW = 16
NC, NS = 2, 16
NROWS = EPAD // 128
RW = NROWS // 32
GR = 8
NG = RW // GR
CHE = GR * 128
NPAD = 50048
STRIPE = NPAD // NS

_mesh = plsc.VectorSubcoreMesh(core_axis_name="c", subcore_axis_name="s",
                               num_cores=NC, num_subcores=NS)
_sc_params = pltpu.CompilerParams(use_tc_tiling_on_sc=False)


def _gather_body(x_hbm, src_hbm, xj_hbm, idx_v, data_v, sem):
    wid = lax.axis_index("c") * NS + lax.axis_index("s")
    r0 = wid * RW
    for g in range(NG):
        rbase = r0 + g * GR
        pltpu.sync_copy(src_hbm.at[pl.ds(rbase, GR)], idx_v)
        cps = [pltpu.async_copy(x_hbm.at[idx_v.at[j]],
                                data_v.at[pl.ds(j * 128, 128)], sem)
               for j in range(GR)]
        for cp in cps:
            cp.wait()
        pltpu.sync_copy(data_v, xj_hbm.at[pl.ds(rbase * 128, CHE)])


def _sc_gather(x_pad, src2d):
    return pl.kernel(
        _gather_body,
        out_type=jax.ShapeDtypeStruct((EPAD, PADW), jnp.float32),
        mesh=_mesh,
        scratch_types=[
            pltpu.VMEM((GR, 128), jnp.int32),
            pltpu.VMEM((CHE, PADW), jnp.float32),
            pltpu.SemaphoreType.DMA,
        ],
        compiler_params=_sc_params,
    )(x_pad, src2d)


def _scatter_body(msg_hbm, dst_hbm, out_hbm, idx_v, data_v, acc_sh, sem):
    c = lax.axis_index("c")
    s = lax.axis_index("s")
    wid = c * NS + s
    r0 = wid * RW

    z16 = jnp.zeros((PADW,), jnp.float32)

    def _zb(i, carry):
        for u in range(8):
            data_v[i * 8 + u, :] = z16
        return carry

    lax.fori_loop(0, CHE // 8, _zb, 0)
    row0 = s * STRIPE
    for zof in range(0, STRIPE - CHE + 1, CHE):
        pltpu.sync_copy(data_v, acc_sh.at[pl.ds(row0 + zof, CHE)])
    zrem = STRIPE % CHE
    if zrem:
        pltpu.sync_copy(data_v.at[pl.ds(0, zrem)],
                        acc_sh.at[pl.ds(row0 + STRIPE - zrem, zrem)])
    plsc.subcore_barrier()

    for g in range(NG):
        rbase = r0 + g * GR
        pltpu.sync_copy(dst_hbm.at[pl.ds(rbase, GR)], idx_v)
        pltpu.sync_copy(msg_hbm.at[pl.ds(rbase * 128, CHE)], data_v)
        cps = [pltpu.async_copy(data_v.at[pl.ds(j * 128, 128)],
                                acc_sh.at[idx_v.at[j]], sem, add=True)
               for j in range(GR)]
        for cp in cps:
            cp.wait()

    plsc.subcore_barrier()
    pltpu.sync_copy(acc_sh.at[pl.ds(row0, STRIPE)],
                    out_hbm.at[c, pl.ds(row0, STRIPE)])


def _sc_scatter(msg, dst2d):
    return pl.kernel(
        _scatter_body,
        out_type=jax.ShapeDtypeStruct((NC, NPAD, PADW), jnp.float32),
        mesh=_mesh,
        scratch_types=[
            pltpu.VMEM((GR, 128), jnp.int32),
            pltpu.VMEM((CHE, PADW), jnp.float32),
            pltpu.VMEM_SHARED((NPAD, PADW), jnp.float32),
            pltpu.SemaphoreType.DMA,
        ],
        compiler_params=_sc_params,
    )(msg, dst2d)


def _pass_body(in_ref, out_ref):
    out_ref[...] = in_ref[...]


def _tc_pass(ea2d):
    return pl.pallas_call(
        _pass_body,
        grid=(_ER // _RB,),
        in_specs=[pl.BlockSpec((_RB, 128), lambda i: (i, 0))],
        out_specs=pl.BlockSpec((_RB, 128), lambda i: (i, 0)),
        out_shape=jax.ShapeDtypeStruct((_ER, 128), jnp.float32),
    )(ea2d)


_NB = 2000
_EB = 8192


def _lstm_body(xp_ref, wih_ref, whh_ref, b_ref, fcw_ref, fcb_ref, out_ref):
    nb = xp_ref.shape[0]
    h = jnp.zeros((nb, HID), jnp.float32)
    c = jnp.zeros((nb, HID), jnp.float32)
    wih = wih_ref[...]
    whh = whh_ref[...]
    b = b_ref[...]
    for t in range(8):
        xt = xp_ref[:, 3 * t:3 * t + 3]
        g = jnp.dot(xt, wih, preferred_element_type=jnp.float32) \
            + jnp.dot(h, whh, preferred_element_type=jnp.float32) + b
        gi = jax.nn.sigmoid(g[:, 0:3])
        gf = jax.nn.sigmoid(g[:, 3:6])
        gg = jnp.tanh(g[:, 6:9])
        go = jax.nn.sigmoid(g[:, 9:12])
        c = gf * c + gi * gg
        h = go * jnp.tanh(c)
    out_ref[...] = jax.nn.relu(
        jnp.dot(h, fcw_ref[...], preferred_element_type=jnp.float32)
        + fcb_ref[...])


def _tc_lstm(xp_t, wih_t, whh_t, bsum, fcw_pad, fcb_pad):
    return pl.pallas_call(
        _lstm_body,
        grid=(N // _NB,),
        in_specs=[
            pl.BlockSpec((_NB, 24), lambda i: (i, 0)),
            pl.BlockSpec((3, 12), lambda i: (0, 0)),
            pl.BlockSpec((3, 12), lambda i: (0, 0)),
            pl.BlockSpec((1, 12), lambda i: (0, 0)),
            pl.BlockSpec((3, PADW), lambda i: (0, 0)),
            pl.BlockSpec((1, PADW), lambda i: (0, 0)),
        ],
        out_specs=pl.BlockSpec((_NB, PADW), lambda i: (i, 0)),
        out_shape=jax.ShapeDtypeStruct((N, PADW), jnp.float32),
    )(xp_t, wih_t, whh_t, bsum, fcw_pad, fcb_pad)


_ER = EPAD // 8
_RB = 1024


def _msg_body(ea_ref, xj_ref, b1_ref, bb1_ref, b2_ref, bb2_ref, bw3_ref,
              bb3_ref, br_ref, bs_ref, be7_ref, out_ref):
    h1 = jax.nn.relu(jnp.dot(ea_ref[...], b1_ref[...],
                             preferred_element_type=jnp.float32) + bb1_ref[...])
    h2 = jax.nn.relu(jnp.dot(h1, b2_ref[...],
                             preferred_element_type=jnp.float32) + bb2_ref[...])
    wflat = jnp.dot(h2, bw3_ref[...],
                    preferred_element_type=jnp.float32) + bb3_ref[...]
    xjrep = jnp.dot(xj_ref[...], br_ref[...],
                    preferred_element_type=jnp.float32)
    prod = wflat * xjrep
    out_ref[...] = jnp.dot(prod, bs_ref[...],
                           preferred_element_type=jnp.float32) + be7_ref[...]


def _tc_msg(ea2d, xj2d, b1, bb1, b2, bb2, bw3, bb3, br, bs, be7):
    return pl.pallas_call(
        _msg_body,
        grid=(_ER // _RB,),
        in_specs=[
            pl.BlockSpec((_RB, 128), lambda i: (i, 0)),
            pl.BlockSpec((_RB, 128), lambda i: (i, 0)),
            pl.BlockSpec((128, 128), lambda i: (0, 0)),
            pl.BlockSpec((1, 128), lambda i: (0, 0)),
            pl.BlockSpec((128, 128), lambda i: (0, 0)),
            pl.BlockSpec((1, 128), lambda i: (0, 0)),
            pl.BlockSpec((128, 392), lambda i: (0, 0)),
            pl.BlockSpec((1, 392), lambda i: (0, 0)),
            pl.BlockSpec((128, 392), lambda i: (0, 0)),
            pl.BlockSpec((392, 128), lambda i: (0, 0)),
            pl.BlockSpec((1, 128), lambda i: (0, 0)),
        ],
        out_specs=pl.BlockSpec((_RB, 128), lambda i: (i, 0)),
        out_shape=jax.ShapeDtypeStruct((_ER, 128), jnp.float32),
    )(ea2d, xj2d, b1, bb1, b2, bb2, bw3, bb3, br, bs, be7)


def _combine_body(acc_ref, x_ref, root_ref, bias_ref, p7_ref, out_ref):
    ssum = acc_ref[0] + acc_ref[1]
    cnt = jnp.maximum(ssum[:, 7:8], 1.0)
    mean16 = jnp.dot(ssum * (1.0 / cnt), p7_ref[...],
                     preferred_element_type=jnp.float32)
    xr = jnp.dot(x_ref[...], root_ref[...],
                 preferred_element_type=jnp.float32) + bias_ref[...]
    out_ref[...] = jax.nn.relu(mean16 + xr)


def _tc_combine(acc, x_pad, root16, bias16, p7):
    return pl.pallas_call(
        _combine_body,
        grid=(N // _NB,),
        in_specs=[
            pl.BlockSpec((2, _NB, PADW), lambda i: (0, i, 0)),
            pl.BlockSpec((_NB, PADW), lambda i: (i, 0)),
            pl.BlockSpec((PADW, PADW), lambda i: (0, 0)),
            pl.BlockSpec((1, PADW), lambda i: (0, 0)),
            pl.BlockSpec((PADW, PADW), lambda i: (0, 0)),
        ],
        out_specs=pl.BlockSpec((_NB, PADW), lambda i: (i, 0)),
        out_shape=jax.ShapeDtypeStruct((N, PADW), jnp.float32),
    )(acc, x_pad, root16, bias16, p7)


def _combine2_body(acc_ref, x_ref, root_ref, bias_ref, p7_ref, fcw_ref,
                   fcb_ref, out_ref):
    ssum = acc_ref[0] + acc_ref[1]
    cnt = jnp.maximum(ssum[:, 7:8], 1.0)
    mean16 = jnp.dot(ssum * (1.0 / cnt), p7_ref[...],
                     preferred_element_type=jnp.float32)
    xr = jnp.dot(x_ref[...], root_ref[...],
                 preferred_element_type=jnp.float32) + bias_ref[...]
    xo = jax.nn.relu(mean16 + xr)
    out_ref[...] = jnp.dot(xo, fcw_ref[...],
                           preferred_element_type=jnp.float32) + fcb_ref[...]


def _tc_combine2(acc, x_pad, root16, bias16, p7, fcw, fcb):
    return pl.pallas_call(
        _combine2_body,
        grid=(N // _NB,),
        in_specs=[
            pl.BlockSpec((2, _NB, PADW), lambda i: (0, i, 0)),
            pl.BlockSpec((_NB, PADW), lambda i: (i, 0)),
            pl.BlockSpec((PADW, PADW), lambda i: (0, 0)),
            pl.BlockSpec((1, PADW), lambda i: (0, 0)),
            pl.BlockSpec((PADW, PADW), lambda i: (0, 0)),
            pl.BlockSpec((PADW, 1), lambda i: (0, 0)),
            pl.BlockSpec((1, 1), lambda i: (0, 0)),
        ],
        out_specs=pl.BlockSpec((_NB, 1), lambda i: (i, 0)),
        out_shape=jax.ShapeDtypeStruct((N, 1), jnp.float32),
    )(acc, x_pad, root16, bias16, p7, fcw, fcb)


def kernel(x_position, edge_index, edge_attr, lstm_Wih, lstm_Whh, lstm_bih,
           lstm_bhh, fcL_W, fcL_b, k_W1, k_b1, k_W2, k_b2, k_W3, k_b3,
           root1, bias1, root2, bias2, fc2_W, fc2_b):
    f32 = jnp.float32
    npd = EPAD - E
    src2d = jnp.concatenate(
        [edge_index[0], jnp.zeros((npd,), jnp.int32)]).reshape(NROWS, 128)
    dst2d = jnp.concatenate(
        [edge_index[1],
         jnp.full((npd,), NPAD - 1, jnp.int32)]).reshape(NROWS, 128)
    ea16 = jnp.concatenate(
        [edge_attr, jnp.zeros((E, PADW - 4), f32)], axis=1)
    ea2d = _tc_pass(jnp.concatenate(
        [ea16, jnp.zeros((npd, PADW), f32)], axis=0).reshape(_ER, 128))
    xp_t = jnp.transpose(x_position, (1, 0, 2)).reshape(N, 24)

    wih_t = lstm_Wih.T
    whh_t = lstm_Whh.T
    bsum = (lstm_bih + lstm_bhh)[None, :]
    fcw_pad = jnp.zeros((3, PADW), f32).at[:, :W].set(fcL_W.T)
    fcb_pad = jnp.zeros((1, PADW), f32).at[0, :W].set(fcL_b)

    i8 = jnp.eye(8, dtype=f32)
    w1t16 = jnp.zeros((16, 16), f32).at[:4, :].set(k_W1.T)
    b1 = jnp.kron(i8, w1t16)
    bb1 = jnp.tile(k_b1, 8)[None, :]
    b2 = jnp.kron(i8, k_W2.T)
    bb2 = jnp.tile(k_b2, 8)[None, :]
    bw3 = jnp.kron(i8, k_W3.T)
    bb3 = jnp.tile(k_b3, 8)[None, :]
    dd = jnp.arange(49) // 7
    rmat16 = jnp.zeros((16, 49), f32).at[:7, :].set(
        (jnp.arange(7)[:, None] == dd[None, :]).astype(f32))
    br = jnp.kron(i8, rmat16)
    oo = jnp.arange(49) % 7
    smat = (oo[:, None] == jnp.arange(PADW)[None, :]).astype(f32)
    bs = jnp.kron(i8, smat)
    be7 = jnp.tile(jnp.zeros((PADW,), f32).at[7].set(1.0), 8)[None, :]
    p7 = jnp.diag((jnp.arange(PADW) < W).astype(f32))

    def _pad_root(r, b):
        r16 = jnp.zeros((PADW, PADW), f32).at[:W, :W].set(r)
        b16 = jnp.zeros((1, PADW), f32).at[0, :W].set(b)
        return r16, b16

    root1p, bias1p = _pad_root(root1, bias1)
    root2p, bias2p = _pad_root(root2, bias2)
    fc2p = jnp.zeros((PADW, 1), f32).at[:W, 0].set(fc2_W[0])
    fc2bp = fc2_b[None, :]

    x1 = _tc_lstm(xp_t, wih_t, whh_t, bsum, fcw_pad, fcb_pad)
    xj1 = _sc_gather(x1, src2d)
    msg1 = _tc_msg(ea2d, xj1.reshape(_ER, 128), b1, bb1, b2, bb2, bw3, bb3,
                   br, bs, be7)
    acc1 = _sc_scatter(msg1.reshape(EPAD, PADW), dst2d)
    x2 = _tc_combine(acc1, x1, root1p, bias1p, p7)
    xj2 = _sc_gather(x2, src2d)
    msg2 = _tc_msg(ea2d, xj2.reshape(_ER, 128), b1, bb1, b2, bb2, bw3, bb3,
                   br, bs, be7)
    acc2 = _sc_scatter(msg2.reshape(EPAD, PADW), dst2d)
    return _tc_combine2(acc2, x2, root2p, bias2p, p7, fc2p, fc2bp)


# speedup vs baseline: 1.0363x; 1.0363x over previous
import jax
import jax.numpy as jnp
from jax import lax
from jax.experimental import pallas as pl
from jax.experimental.pallas import tpu as pltpu
from jax.experimental.pallas import tpu_sc as plsc

N = 50000
E = 800000
EPAD = 819200
HID = 3
W = 7
PADW = 16
NC, NS = 2, 16
NROWS = EPAD // 128
RW = NROWS // 32
GR = 8
NG = RW // GR
CHE = GR * 128
NPAD = 50048
STRIPE = NPAD // NS

_mesh = plsc.VectorSubcoreMesh(core_axis_name="c", subcore_axis_name="s",
                               num_cores=NC, num_subcores=NS)
_sc_params = pltpu.CompilerParams(use_tc_tiling_on_sc=False)


def _gather_body(x_hbm, src_hbm, xj_hbm, idx_v, data_v, semi, semg, semo):
    wid = lax.axis_index("c") * NS + lax.axis_index("s")
    r0 = wid * RW
    pltpu.async_copy(src_hbm.at[pl.ds(r0, GR)], idx_v.at[0], semi).wait()
    idx_cp = pltpu.async_copy(src_hbm.at[pl.ds(r0 + GR, GR)],
                              idx_v.at[1], semi)
    out_cps = [None, None]
    for g in range(NG):
        b = g % 2
        rbase = r0 + g * GR
        if out_cps[b] is not None:
            out_cps[b].wait()
        cps = [pltpu.async_copy(x_hbm.at[idx_v.at[b, j]],
                                data_v.at[b, pl.ds(j * 128, 128)], semg)
               for j in range(GR)]
        if g + 1 < NG:
            idx_cp.wait()
        for cp in cps:
            cp.wait()
        if g + 2 < NG:
            idx_cp = pltpu.async_copy(
                src_hbm.at[pl.ds(rbase + 2 * GR, GR)], idx_v.at[b], semi)
        out_cps[b] = pltpu.async_copy(data_v.at[b],
                                      xj_hbm.at[pl.ds(rbase * 128, CHE)],
                                      semo)
    out_cps[0].wait()
    out_cps[1].wait()


def _sc_gather(x_pad, src2d):
    return pl.kernel(
        _gather_body,
        out_type=jax.ShapeDtypeStruct((EPAD, PADW), jnp.float32),
        mesh=_mesh,
        scratch_types=[
            pltpu.VMEM((2, GR, 128), jnp.int32),
            pltpu.VMEM((2, CHE, PADW), jnp.float32),
            pltpu.SemaphoreType.DMA,
            pltpu.SemaphoreType.DMA,
            pltpu.SemaphoreType.DMA,
        ],
        compiler_params=_sc_params,
    )(x_pad, src2d)


def _scatter_body(msg_hbm, dst_hbm, out_hbm, idx_v, data_v, acc_sh, sem):
    c = lax.axis_index("c")
    s = lax.axis_index("s")
    wid = c * NS + s
    r0 = wid * RW

    z16 = jnp.zeros((PADW,), jnp.float32)

    def _zb(i, carry):
        for u in range(8):
            data_v[i * 8 + u, :] = z16
        return carry

    lax.fori_loop(0, CHE // 8, _zb, 0)
    row0 = s * STRIPE
    for zof in range(0, STRIPE - CHE + 1, CHE):
        pltpu.sync_copy(data_v, acc_sh.at[pl.ds(row0 + zof, CHE)])
    zrem = STRIPE % CHE
    if zrem:
        pltpu.sync_copy(data_v.at[pl.ds(0, zrem)],
                        acc_sh.at[pl.ds(row0 + STRIPE - zrem, zrem)])
    plsc.subcore_barrier()

    for g in range(NG):
        rbase = r0 + g * GR
        pltpu.sync_copy(dst_hbm.at[pl.ds(rbase, GR)], idx_v)
        pltpu.sync_copy(msg_hbm.at[pl.ds(rbase * 128, CHE)], data_v)
        cps = [pltpu.async_copy(data_v.at[pl.ds(j * 128, 128)],
                                acc_sh.at[idx_v.at[j]], sem, add=True)
               for j in range(GR)]
        for cp in cps:
            cp.wait()

    plsc.subcore_barrier()
    pltpu.sync_copy(acc_sh.at[pl.ds(row0, STRIPE)],
                    out_hbm.at[c, pl.ds(row0, STRIPE)])


def _sc_scatter(msg, dst2d):
    return pl.kernel(
        _scatter_body,
        out_type=jax.ShapeDtypeStruct((NC, NPAD, PADW), jnp.float32),
        mesh=_mesh,
        scratch_types=[
            pltpu.VMEM((GR, 128), jnp.int32),
            pltpu.VMEM((CHE, PADW), jnp.float32),
            pltpu.VMEM_SHARED((NPAD, PADW), jnp.float32),
            pltpu.SemaphoreType.DMA,
        ],
        compiler_params=_sc_params,
    )(msg, dst2d)


_NB = 2000
_EB = 8192


def _lstm_body(xp_ref, wih_ref, whh_ref, b_ref, fcw_ref, fcb_ref, out_ref):
    nb = xp_ref.shape[0]
    h = jnp.zeros((nb, HID), jnp.float32)
    c = jnp.zeros((nb, HID), jnp.float32)
    wih = wih_ref[...]
    whh = whh_ref[...]
    b = b_ref[...]
    for t in range(8):
        xt = xp_ref[:, 3 * t:3 * t + 3]
        g = jnp.dot(xt, wih, preferred_element_type=jnp.float32) \
            + jnp.dot(h, whh, preferred_element_type=jnp.float32) + b
        gi = jax.nn.sigmoid(g[:, 0:3])
        gf = jax.nn.sigmoid(g[:, 3:6])
        gg = jnp.tanh(g[:, 6:9])
        go = jax.nn.sigmoid(g[:, 9:12])
        c = gf * c + gi * gg
        h = go * jnp.tanh(c)
    out_ref[...] = jax.nn.relu(
        jnp.dot(h, fcw_ref[...], preferred_element_type=jnp.float32)
        + fcb_ref[...])


def _tc_lstm(xp_t, wih_t, whh_t, bsum, fcw_pad, fcb_pad):
    return pl.pallas_call(
        _lstm_body,
        grid=(N // _NB,),
        in_specs=[
            pl.BlockSpec((_NB, 24), lambda i: (i, 0)),
            pl.BlockSpec((3, 12), lambda i: (0, 0)),
            pl.BlockSpec((3, 12), lambda i: (0, 0)),
            pl.BlockSpec((1, 12), lambda i: (0, 0)),
            pl.BlockSpec((3, PADW), lambda i: (0, 0)),
            pl.BlockSpec((1, PADW), lambda i: (0, 0)),
        ],
        out_specs=pl.BlockSpec((_NB, PADW), lambda i: (i, 0)),
        out_shape=jax.ShapeDtypeStruct((N, PADW), jnp.float32),
    )(xp_t, wih_t, whh_t, bsum, fcw_pad, fcb_pad)


_ER = EPAD // 8
_RB = 1024


def _msg_body(ea_ref, xj_ref, b1_ref, bb1_ref, b2_ref, bb2_ref, bw3_ref,
              bb3_ref, br_ref, bs_ref, be7_ref, out_ref):
    h1 = jax.nn.relu(jnp.dot(ea_ref[...], b1_ref[...],
                             preferred_element_type=jnp.float32) + bb1_ref[...])
    h2 = jax.nn.relu(jnp.dot(h1, b2_ref[...],
                             preferred_element_type=jnp.float32) + bb2_ref[...])
    wflat = jnp.dot(h2, bw3_ref[...],
                    preferred_element_type=jnp.float32) + bb3_ref[...]
    xjrep = jnp.dot(xj_ref[...], br_ref[...],
                    preferred_element_type=jnp.float32)
    prod = wflat * xjrep
    out_ref[...] = jnp.dot(prod, bs_ref[...],
                           preferred_element_type=jnp.float32) + be7_ref[...]


def _tc_msg(ea2d, xj2d, b1, bb1, b2, bb2, bw3, bb3, br, bs, be7):
    return pl.pallas_call(
        _msg_body,
        grid=(_ER // _RB,),
        in_specs=[
            pl.BlockSpec((_RB, 128), lambda i: (i, 0)),
            pl.BlockSpec((_RB, 128), lambda i: (i, 0)),
            pl.BlockSpec((128, 128), lambda i: (0, 0)),
            pl.BlockSpec((1, 128), lambda i: (0, 0)),
            pl.BlockSpec((128, 128), lambda i: (0, 0)),
            pl.BlockSpec((1, 128), lambda i: (0, 0)),
            pl.BlockSpec((128, 392), lambda i: (0, 0)),
            pl.BlockSpec((1, 392), lambda i: (0, 0)),
            pl.BlockSpec((128, 392), lambda i: (0, 0)),
            pl.BlockSpec((392, 128), lambda i: (0, 0)),
            pl.BlockSpec((1, 128), lambda i: (0, 0)),
        ],
        out_specs=pl.BlockSpec((_RB, 128), lambda i: (i, 0)),
        out_shape=jax.ShapeDtypeStruct((_ER, 128), jnp.float32),
    )(ea2d, xj2d, b1, bb1, b2, bb2, bw3, bb3, br, bs, be7)


def _combine_body(acc_ref, x_ref, root_ref, bias_ref, p7_ref, out_ref):
    ssum = acc_ref[0] + acc_ref[1]
    cnt = jnp.maximum(ssum[:, 7:8], 1.0)
    mean16 = jnp.dot(ssum * (1.0 / cnt), p7_ref[...],
                     preferred_element_type=jnp.float32)
    xr = jnp.dot(x_ref[...], root_ref[...],
                 preferred_element_type=jnp.float32) + bias_ref[...]
    out_ref[...] = jax.nn.relu(mean16 + xr)


def _tc_combine(acc, x_pad, root16, bias16, p7):
    return pl.pallas_call(
        _combine_body,
        grid=(N // _NB,),
        in_specs=[
            pl.BlockSpec((2, _NB, PADW), lambda i: (0, i, 0)),
            pl.BlockSpec((_NB, PADW), lambda i: (i, 0)),
            pl.BlockSpec((PADW, PADW), lambda i: (0, 0)),
            pl.BlockSpec((1, PADW), lambda i: (0, 0)),
            pl.BlockSpec((PADW, PADW), lambda i: (0, 0)),
        ],
        out_specs=pl.BlockSpec((_NB, PADW), lambda i: (i, 0)),
        out_shape=jax.ShapeDtypeStruct((N, PADW), jnp.float32),
    )(acc, x_pad, root16, bias16, p7)


def _combine2_body(acc_ref, x_ref, root_ref, bias_ref, p7_ref, fcw_ref,
                   fcb_ref, out_ref):
    ssum = acc_ref[0] + acc_ref[1]
    cnt = jnp.maximum(ssum[:, 7:8], 1.0)
    mean16 = jnp.dot(ssum * (1.0 / cnt), p7_ref[...],
                     preferred_element_type=jnp.float32)
    xr = jnp.dot(x_ref[...], root_ref[...],
                 preferred_element_type=jnp.float32) + bias_ref[...]
    xo = jax.nn.relu(mean16 + xr)
    out_ref[...] = jnp.dot(xo, fcw_ref[...],
                           preferred_element_type=jnp.float32) + fcb_ref[...]


def _tc_combine2(acc, x_pad, root16, bias16, p7, fcw, fcb):
    return pl.pallas_call(
        _combine2_body,
        grid=(N // _NB,),
        in_specs=[
            pl.BlockSpec((2, _NB, PADW), lambda i: (0, i, 0)),
            pl.BlockSpec((_NB, PADW), lambda i: (i, 0)),
            pl.BlockSpec((PADW, PADW), lambda i: (0, 0)),
            pl.BlockSpec((1, PADW), lambda i: (0, 0)),
            pl.BlockSpec((PADW, PADW), lambda i: (0, 0)),
            pl.BlockSpec((PADW, 1), lambda i: (0, 0)),
            pl.BlockSpec((1, 1), lambda i: (0, 0)),
        ],
        out_specs=pl.BlockSpec((_NB, 1), lambda i: (i, 0)),
        out_shape=jax.ShapeDtypeStruct((N, 1), jnp.float32),
    )(acc, x_pad, root16, bias16, p7, fcw, fcb)


def kernel(x_position, edge_index, edge_attr, lstm_Wih, lstm_Whh, lstm_bih,
           lstm_bhh, fcL_W, fcL_b, k_W1, k_b1, k_W2, k_b2, k_W3, k_b3,
           root1, bias1, root2, bias2, fc2_W, fc2_b):
    f32 = jnp.float32
    npd = EPAD - E
    src2d = jnp.concatenate(
        [edge_index[0], jnp.zeros((npd,), jnp.int32)]).reshape(NROWS, 128)
    dst2d = jnp.concatenate(
        [edge_index[1],
         jnp.full((npd,), NPAD - 1, jnp.int32)]).reshape(NROWS, 128)
    ea16 = jnp.concatenate(
        [edge_attr, jnp.zeros((E, PADW - 4), f32)], axis=1)
    ea2d = jnp.concatenate(
        [ea16, jnp.zeros((npd, PADW), f32)], axis=0).reshape(_ER, 128)
    xp_t = jnp.transpose(x_position, (1, 0, 2)).reshape(N, 24)

    wih_t = lstm_Wih.T
    whh_t = lstm_Whh.T
    bsum = (lstm_bih + lstm_bhh)[None, :]
    fcw_pad = jnp.zeros((3, PADW), f32).at[:, :W].set(fcL_W.T)
    fcb_pad = jnp.zeros((1, PADW), f32).at[0, :W].set(fcL_b)

    i8 = jnp.eye(8, dtype=f32)
    w1t16 = jnp.zeros((16, 16), f32).at[:4, :].set(k_W1.T)
    b1 = jnp.kron(i8, w1t16)
    bb1 = jnp.tile(k_b1, 8)[None, :]
    b2 = jnp.kron(i8, k_W2.T)
    bb2 = jnp.tile(k_b2, 8)[None, :]
    bw3 = jnp.kron(i8, k_W3.T)
    bb3 = jnp.tile(k_b3, 8)[None, :]
    dd = jnp.arange(49) // 7
    rmat16 = jnp.zeros((16, 49), f32).at[:7, :].set(
        (jnp.arange(7)[:, None] == dd[None, :]).astype(f32))
    br = jnp.kron(i8, rmat16)
    oo = jnp.arange(49) % 7
    smat = (oo[:, None] == jnp.arange(PADW)[None, :]).astype(f32)
    bs = jnp.kron(i8, smat)
    be7 = jnp.tile(jnp.zeros((PADW,), f32).at[7].set(1.0), 8)[None, :]
    p7 = jnp.diag((jnp.arange(PADW) < W).astype(f32))

    def _pad_root(r, b):
        r16 = jnp.zeros((PADW, PADW), f32).at[:W, :W].set(r)
        b16 = jnp.zeros((1, PADW), f32).at[0, :W].set(b)
        return r16, b16

    root1p, bias1p = _pad_root(root1, bias1)
    root2p, bias2p = _pad_root(root2, bias2)
    fc2p = jnp.zeros((PADW, 1), f32).at[:W, 0].set(fc2_W[0])
    fc2bp = fc2_b[None, :]

    x1 = _tc_lstm(xp_t, wih_t, whh_t, bsum, fcw_pad, fcb_pad)
    xj1 = _sc_gather(x1, src2d)
    msg1 = _tc_msg(ea2d, xj1.reshape(_ER, 128), b1, bb1, b2, bb2, bw3, bb3,
                   br, bs, be7)
    acc1 = _sc_scatter(msg1.reshape(EPAD, PADW), dst2d)
    x2 = _tc_combine(acc1, x1, root1p, bias1p, p7)
    xj2 = _sc_gather(x2, src2d)
    msg2 = _tc_msg(ea2d, xj2.reshape(_ER, 128), b1, bb1, b2, bb2, bw3, bb3,
                   br, bs, be7)
    acc2 = _sc_scatter(msg2.reshape(EPAD, PADW), dst2d)
    return _tc_combine2(acc2, x2, root2p, bias2p, p7, fc2p, fc2bp)


# speedup vs baseline: 1.1697x; 1.1288x over previous
import jax
import jax.numpy as jnp
from jax import lax
from jax.experimental import pallas as pl
from jax.experimental.pallas import tpu as pltpu
from jax.experimental.pallas import tpu_sc as plsc

N = 50000
E = 800000
EPAD = 819200
HID = 3
W = 7
PADW = 16
NC, NS = 2, 16
NROWS = EPAD // 128
RW = NROWS // 32
GR = 8
NG = RW // GR
CHE = GR * 128
NPAD = 50048
STRIPE = NPAD // NS

_mesh = plsc.VectorSubcoreMesh(core_axis_name="c", subcore_axis_name="s",
                               num_cores=NC, num_subcores=NS)
_sc_params = pltpu.CompilerParams(use_tc_tiling_on_sc=False)


def _gather_body(x_hbm, src_hbm, xj_hbm, idx_v, data_v, x_sh, semi, semg,
                 semo):
    s = lax.axis_index("s")
    wid = lax.axis_index("c") * NS + s
    r0 = wid * RW
    pltpu.sync_copy(x_hbm.at[pl.ds(s * (N // NS), N // NS)],
                    x_sh.at[pl.ds(s * (N // NS), N // NS)])
    plsc.subcore_barrier()
    pltpu.async_copy(src_hbm.at[pl.ds(r0, GR)], idx_v.at[0], semi).wait()
    idx_cp = pltpu.async_copy(src_hbm.at[pl.ds(r0 + GR, GR)],
                              idx_v.at[1], semi)
    out_cps = [None, None]
    for g in range(NG):
        b = g % 2
        rbase = r0 + g * GR
        if out_cps[b] is not None:
            out_cps[b].wait()
        cps = [pltpu.async_copy(x_sh.at[idx_v.at[b, j]],
                                data_v.at[b, pl.ds(j * 128, 128)], semg)
               for j in range(GR)]
        if g + 1 < NG:
            idx_cp.wait()
        for cp in cps:
            cp.wait()
        if g + 2 < NG:
            idx_cp = pltpu.async_copy(
                src_hbm.at[pl.ds(rbase + 2 * GR, GR)], idx_v.at[b], semi)
        out_cps[b] = pltpu.async_copy(data_v.at[b],
                                      xj_hbm.at[pl.ds(rbase * 128, CHE)],
                                      semo)
    out_cps[0].wait()
    out_cps[1].wait()


def _sc_gather(x_pad, src2d):
    return pl.kernel(
        _gather_body,
        out_type=jax.ShapeDtypeStruct((EPAD, PADW), jnp.float32),
        mesh=_mesh,
        scratch_types=[
            pltpu.VMEM((2, GR, 128), jnp.int32),
            pltpu.VMEM((2, CHE, PADW), jnp.float32),
            pltpu.VMEM_SHARED((N, PADW), jnp.float32),
            pltpu.SemaphoreType.DMA,
            pltpu.SemaphoreType.DMA,
            pltpu.SemaphoreType.DMA,
        ],
        compiler_params=_sc_params,
    )(x_pad, src2d)


def _scatter_body(msg_hbm, dst_hbm, out_hbm, idx_v, data_v, acc_sh, sem):
    c = lax.axis_index("c")
    s = lax.axis_index("s")
    wid = c * NS + s
    r0 = wid * RW

    z16 = jnp.zeros((PADW,), jnp.float32)

    def _zb(i, carry):
        for u in range(8):
            data_v[i * 8 + u, :] = z16
        return carry

    lax.fori_loop(0, CHE // 8, _zb, 0)
    row0 = s * STRIPE
    for zof in range(0, STRIPE - CHE + 1, CHE):
        pltpu.sync_copy(data_v, acc_sh.at[pl.ds(row0 + zof, CHE)])
    zrem = STRIPE % CHE
    if zrem:
        pltpu.sync_copy(data_v.at[pl.ds(0, zrem)],
                        acc_sh.at[pl.ds(row0 + STRIPE - zrem, zrem)])
    plsc.subcore_barrier()

    for g in range(NG):
        rbase = r0 + g * GR
        pltpu.sync_copy(dst_hbm.at[pl.ds(rbase, GR)], idx_v)
        pltpu.sync_copy(msg_hbm.at[pl.ds(rbase * 128, CHE)], data_v)
        cps = [pltpu.async_copy(data_v.at[pl.ds(j * 128, 128)],
                                acc_sh.at[idx_v.at[j]], sem, add=True)
               for j in range(GR)]
        for cp in cps:
            cp.wait()

    plsc.subcore_barrier()
    pltpu.sync_copy(acc_sh.at[pl.ds(row0, STRIPE)],
                    out_hbm.at[c, pl.ds(row0, STRIPE)])


def _sc_scatter(msg, dst2d):
    return pl.kernel(
        _scatter_body,
        out_type=jax.ShapeDtypeStruct((NC, NPAD, PADW), jnp.float32),
        mesh=_mesh,
        scratch_types=[
            pltpu.VMEM((GR, 128), jnp.int32),
            pltpu.VMEM((CHE, PADW), jnp.float32),
            pltpu.VMEM_SHARED((NPAD, PADW), jnp.float32),
            pltpu.SemaphoreType.DMA,
        ],
        compiler_params=_sc_params,
    )(msg, dst2d)


_NB = 2000
_EB = 8192


def _lstm_body(xp_ref, wih_ref, whh_ref, b_ref, fcw_ref, fcb_ref, out_ref):
    nb = xp_ref.shape[0]
    h = jnp.zeros((nb, HID), jnp.float32)
    c = jnp.zeros((nb, HID), jnp.float32)
    wih = wih_ref[...]
    whh = whh_ref[...]
    b = b_ref[...]
    for t in range(8):
        xt = xp_ref[:, 3 * t:3 * t + 3]
        g = jnp.dot(xt, wih, preferred_element_type=jnp.float32) \
            + jnp.dot(h, whh, preferred_element_type=jnp.float32) + b
        gi = jax.nn.sigmoid(g[:, 0:3])
        gf = jax.nn.sigmoid(g[:, 3:6])
        gg = jnp.tanh(g[:, 6:9])
        go = jax.nn.sigmoid(g[:, 9:12])
        c = gf * c + gi * gg
        h = go * jnp.tanh(c)
    out_ref[...] = jax.nn.relu(
        jnp.dot(h, fcw_ref[...], preferred_element_type=jnp.float32)
        + fcb_ref[...])


def _tc_lstm(xp_t, wih_t, whh_t, bsum, fcw_pad, fcb_pad):
    return pl.pallas_call(
        _lstm_body,
        grid=(N // _NB,),
        in_specs=[
            pl.BlockSpec((_NB, 24), lambda i: (i, 0)),
            pl.BlockSpec((3, 12), lambda i: (0, 0)),
            pl.BlockSpec((3, 12), lambda i: (0, 0)),
            pl.BlockSpec((1, 12), lambda i: (0, 0)),
            pl.BlockSpec((3, PADW), lambda i: (0, 0)),
            pl.BlockSpec((1, PADW), lambda i: (0, 0)),
        ],
        out_specs=pl.BlockSpec((_NB, PADW), lambda i: (i, 0)),
        out_shape=jax.ShapeDtypeStruct((N, PADW), jnp.float32),
    )(xp_t, wih_t, whh_t, bsum, fcw_pad, fcb_pad)


_ER = EPAD // 8
_RB = 1024


def _msg_body(ea_ref, xj_ref, b1_ref, bb1_ref, b2_ref, bb2_ref, bw3_ref,
              bb3_ref, br_ref, bs_ref, be7_ref, out_ref):
    h1 = jax.nn.relu(jnp.dot(ea_ref[...], b1_ref[...],
                             preferred_element_type=jnp.float32) + bb1_ref[...])
    h2 = jax.nn.relu(jnp.dot(h1, b2_ref[...],
                             preferred_element_type=jnp.float32) + bb2_ref[...])
    wflat = jnp.dot(h2, bw3_ref[...],
                    preferred_element_type=jnp.float32) + bb3_ref[...]
    xjrep = jnp.dot(xj_ref[...], br_ref[...],
                    preferred_element_type=jnp.float32)
    prod = wflat * xjrep
    out_ref[...] = jnp.dot(prod, bs_ref[...],
                           preferred_element_type=jnp.float32) + be7_ref[...]


def _tc_msg(ea2d, xj2d, b1, bb1, b2, bb2, bw3, bb3, br, bs, be7):
    return pl.pallas_call(
        _msg_body,
        grid=(_ER // _RB,),
        in_specs=[
            pl.BlockSpec((_RB, 128), lambda i: (i, 0)),
            pl.BlockSpec((_RB, 128), lambda i: (i, 0)),
            pl.BlockSpec((128, 128), lambda i: (0, 0)),
            pl.BlockSpec((1, 128), lambda i: (0, 0)),
            pl.BlockSpec((128, 128), lambda i: (0, 0)),
            pl.BlockSpec((1, 128), lambda i: (0, 0)),
            pl.BlockSpec((128, 392), lambda i: (0, 0)),
            pl.BlockSpec((1, 392), lambda i: (0, 0)),
            pl.BlockSpec((128, 392), lambda i: (0, 0)),
            pl.BlockSpec((392, 128), lambda i: (0, 0)),
            pl.BlockSpec((1, 128), lambda i: (0, 0)),
        ],
        out_specs=pl.BlockSpec((_RB, 128), lambda i: (i, 0)),
        out_shape=jax.ShapeDtypeStruct((_ER, 128), jnp.float32),
    )(ea2d, xj2d, b1, bb1, b2, bb2, bw3, bb3, br, bs, be7)


def _combine_body(acc_ref, x_ref, root_ref, bias_ref, p7_ref, out_ref):
    ssum = acc_ref[0] + acc_ref[1]
    cnt = jnp.maximum(ssum[:, 7:8], 1.0)
    mean16 = jnp.dot(ssum * (1.0 / cnt), p7_ref[...],
                     preferred_element_type=jnp.float32)
    xr = jnp.dot(x_ref[...], root_ref[...],
                 preferred_element_type=jnp.float32) + bias_ref[...]
    out_ref[...] = jax.nn.relu(mean16 + xr)


def _tc_combine(acc, x_pad, root16, bias16, p7):
    return pl.pallas_call(
        _combine_body,
        grid=(N // _NB,),
        in_specs=[
            pl.BlockSpec((2, _NB, PADW), lambda i: (0, i, 0)),
            pl.BlockSpec((_NB, PADW), lambda i: (i, 0)),
            pl.BlockSpec((PADW, PADW), lambda i: (0, 0)),
            pl.BlockSpec((1, PADW), lambda i: (0, 0)),
            pl.BlockSpec((PADW, PADW), lambda i: (0, 0)),
        ],
        out_specs=pl.BlockSpec((_NB, PADW), lambda i: (i, 0)),
        out_shape=jax.ShapeDtypeStruct((N, PADW), jnp.float32),
    )(acc, x_pad, root16, bias16, p7)


def _combine2_body(acc_ref, x_ref, root_ref, bias_ref, p7_ref, fcw_ref,
                   fcb_ref, out_ref):
    ssum = acc_ref[0] + acc_ref[1]
    cnt = jnp.maximum(ssum[:, 7:8], 1.0)
    mean16 = jnp.dot(ssum * (1.0 / cnt), p7_ref[...],
                     preferred_element_type=jnp.float32)
    xr = jnp.dot(x_ref[...], root_ref[...],
                 preferred_element_type=jnp.float32) + bias_ref[...]
    xo = jax.nn.relu(mean16 + xr)
    out_ref[...] = jnp.dot(xo, fcw_ref[...],
                           preferred_element_type=jnp.float32) + fcb_ref[...]


def _tc_combine2(acc, x_pad, root16, bias16, p7, fcw, fcb):
    return pl.pallas_call(
        _combine2_body,
        grid=(N // _NB,),
        in_specs=[
            pl.BlockSpec((2, _NB, PADW), lambda i: (0, i, 0)),
            pl.BlockSpec((_NB, PADW), lambda i: (i, 0)),
            pl.BlockSpec((PADW, PADW), lambda i: (0, 0)),
            pl.BlockSpec((1, PADW), lambda i: (0, 0)),
            pl.BlockSpec((PADW, PADW), lambda i: (0, 0)),
            pl.BlockSpec((PADW, 1), lambda i: (0, 0)),
            pl.BlockSpec((1, 1), lambda i: (0, 0)),
        ],
        out_specs=pl.BlockSpec((_NB, 1), lambda i: (i, 0)),
        out_shape=jax.ShapeDtypeStruct((N, 1), jnp.float32),
    )(acc, x_pad, root16, bias16, p7, fcw, fcb)


def kernel(x_position, edge_index, edge_attr, lstm_Wih, lstm_Whh, lstm_bih,
           lstm_bhh, fcL_W, fcL_b, k_W1, k_b1, k_W2, k_b2, k_W3, k_b3,
           root1, bias1, root2, bias2, fc2_W, fc2_b):
    f32 = jnp.float32
    npd = EPAD - E
    src2d = jnp.concatenate(
        [edge_index[0], jnp.zeros((npd,), jnp.int32)]).reshape(NROWS, 128)
    dst2d = jnp.concatenate(
        [edge_index[1],
         jnp.full((npd,), NPAD - 1, jnp.int32)]).reshape(NROWS, 128)
    ea16 = jnp.concatenate(
        [edge_attr, jnp.zeros((E, PADW - 4), f32)], axis=1)
    ea2d = jnp.concatenate(
        [ea16, jnp.zeros((npd, PADW), f32)], axis=0).reshape(_ER, 128)
    xp_t = jnp.transpose(x_position, (1, 0, 2)).reshape(N, 24)

    wih_t = lstm_Wih.T
    whh_t = lstm_Whh.T
    bsum = (lstm_bih + lstm_bhh)[None, :]
    fcw_pad = jnp.zeros((3, PADW), f32).at[:, :W].set(fcL_W.T)
    fcb_pad = jnp.zeros((1, PADW), f32).at[0, :W].set(fcL_b)

    i8 = jnp.eye(8, dtype=f32)
    w1t16 = jnp.zeros((16, 16), f32).at[:4, :].set(k_W1.T)
    b1 = jnp.kron(i8, w1t16)
    bb1 = jnp.tile(k_b1, 8)[None, :]
    b2 = jnp.kron(i8, k_W2.T)
    bb2 = jnp.tile(k_b2, 8)[None, :]
    bw3 = jnp.kron(i8, k_W3.T)
    bb3 = jnp.tile(k_b3, 8)[None, :]
    dd = jnp.arange(49) // 7
    rmat16 = jnp.zeros((16, 49), f32).at[:7, :].set(
        (jnp.arange(7)[:, None] == dd[None, :]).astype(f32))
    br = jnp.kron(i8, rmat16)
    oo = jnp.arange(49) % 7
    smat = (oo[:, None] == jnp.arange(PADW)[None, :]).astype(f32)
    bs = jnp.kron(i8, smat)
    be7 = jnp.tile(jnp.zeros((PADW,), f32).at[7].set(1.0), 8)[None, :]
    p7 = jnp.diag((jnp.arange(PADW) < W).astype(f32))

    def _pad_root(r, b):
        r16 = jnp.zeros((PADW, PADW), f32).at[:W, :W].set(r)
        b16 = jnp.zeros((1, PADW), f32).at[0, :W].set(b)
        return r16, b16

    root1p, bias1p = _pad_root(root1, bias1)
    root2p, bias2p = _pad_root(root2, bias2)
    fc2p = jnp.zeros((PADW, 1), f32).at[:W, 0].set(fc2_W[0])
    fc2bp = fc2_b[None, :]

    x1 = _tc_lstm(xp_t, wih_t, whh_t, bsum, fcw_pad, fcb_pad)
    xj1 = _sc_gather(x1, src2d)
    msg1 = _tc_msg(ea2d, xj1.reshape(_ER, 128), b1, bb1, b2, bb2, bw3, bb3,
                   br, bs, be7)
    acc1 = _sc_scatter(msg1.reshape(EPAD, PADW), dst2d)
    x2 = _tc_combine(acc1, x1, root1p, bias1p, p7)
    xj2 = _sc_gather(x2, src2d)
    msg2 = _tc_msg(ea2d, xj2.reshape(_ER, 128), b1, bb1, b2, bb2, bw3, bb3,
                   br, bs, be7)
    acc2 = _sc_scatter(msg2.reshape(EPAD, PADW), dst2d)
    return _tc_combine2(acc2, x2, root2p, bias2p, p7, fc2p, fc2bp)
